# baseline (speedup 1.0000x reference)
import functools

import jax
import jax.numpy as jnp
from jax import lax
from jax.experimental import pallas as pl
from jax.experimental.pallas import tpu as pltpu
from jax.experimental.pallas import tpu_sc as plsc


def _prep_body(idx_ref, src_ref, rowT_ref, srcT_ref):
    rowT_ref[...] = idx_ref[...].T
    srcT_ref[...] = src_ref[...].T


_GROUPS = 8
_GCOLS = 16
_CHUNK = 4096


def _make_sc_scatter(m, d, b, nw):
    mesh = plsc.VectorSubcoreMesh(core_axis_name="c", subcore_axis_name="s")
    bands = 20
    rows_band = m // bands
    n_tiles = _GROUPS * bands
    tiles_per_w = n_tiles // nw
    chunks_per_col = b // _CHUNK
    n_chunks = _GCOLS * chunks_per_col
    nv = _CHUNK // 16

    @functools.partial(
        pl.kernel,
        mesh=mesh,
        out_type=jax.ShapeDtypeStruct((m, d), jnp.float32),
        compiler_params=pltpu.CompilerParams(
            needs_layout_passes=False, use_tc_tiling_on_sc=False,
            disable_bounds_checks=True),
        scratch_types=[
            pltpu.VMEM((rows_band, _GCOLS), jnp.float32),
            pltpu.VMEM((_CHUNK,), jnp.int32),
            pltpu.VMEM((_CHUNK,), jnp.int32),
            pltpu.VMEM((_CHUNK,), jnp.float32),
            pltpu.VMEM((_CHUNK,), jnp.float32),
            pltpu.SemaphoreType.DMA,
            pltpu.SemaphoreType.DMA,
            pltpu.SemaphoreType.DMA,
            pltpu.SemaphoreType.DMA,
        ],
    )
    def _sc(x_ref, rowT_ref, srcT_ref, out_ref,
            tile_v, rb0, rb1, vb0, vb1, sr0, sr1, sv0, sv1):
        c = lax.axis_index("c")
        s = lax.axis_index("s")
        w = s * 2 + c

        def chunk_off(g, q):
            col = q // chunks_per_col
            hc = lax.rem(q, chunks_per_col)
            return (g * _GCOLS + col) * b + hc * _CHUNK

        for i in range(tiles_per_w):
            tid = w + nw * i
            g = lax.rem(tid, _GROUPS)
            band = tid // _GROUPS
            b0 = band * rows_band
            c0 = g * _GCOLS

            if True:
                pltpu.sync_copy(
                    x_ref.at[pl.ds(b0, rows_band), pl.ds(c0, _GCOLS)], tile_v)

            pltpu.async_copy(rowT_ref.at[pl.ds(chunk_off(g, 0), _CHUNK)],
                             rb0, sr0)
            pltpu.async_copy(srcT_ref.at[pl.ds(chunk_off(g, 0), _CHUNK)],
                             vb0, sv0)
            pltpu.async_copy(rowT_ref.at[pl.ds(chunk_off(g, 1), _CHUNK)],
                             rb1, sr1)
            pltpu.async_copy(srcT_ref.at[pl.ds(chunk_off(g, 1), _CHUNK)],
                             vb1, sv1)

            @pl.loop(0, n_chunks // 2)
            def _pair(p):
                for ph, (rb, vb, sr, sv) in enumerate(
                        ((rb0, vb0, sr0, sv0), (rb1, vb1, sr1, sv1))):
                    q = 2 * p + ph
                    col = q // chunks_per_col
                    pltpu.make_async_copy(
                        rowT_ref.at[pl.ds(0, _CHUNK)], rb, sr).wait()
                    pltpu.make_async_copy(
                        srcT_ref.at[pl.ds(0, _CHUNK)], vb, sv).wait()

                    @pl.loop(0, nv, unroll=8)
                    def _vec(t):
                        sl = pl.ds(t * 16, 16)
                        r16 = rb[sl]
                        inb = (r16 >= b0) & (r16 < b0 + rows_band)
                        rr = r16 - b0
                        c16 = jnp.full((16,), col, jnp.int32)
                        plsc.store_scatter(
                            tile_v, [rr, c16], vb[sl], mask=inb)

                    nq = q + 2

                    @pl.when(nq < n_chunks)
                    def _():
                        off = chunk_off(g, nq)
                        pltpu.async_copy(
                            rowT_ref.at[pl.ds(off, _CHUNK)], rb, sr)
                        pltpu.async_copy(
                            srcT_ref.at[pl.ds(off, _CHUNK)], vb, sv)

            if True:
                pltpu.sync_copy(
                    tile_v, out_ref.at[pl.ds(b0, rows_band), pl.ds(c0, _GCOLS)])

    return _sc


def kernel(x, index, src):
    m, d = x.shape
    b = index.shape[0]
    assert d == 128 and b % 128 == 0

    idx32 = index.astype(jnp.int32)

    bt = 2048
    prep = pl.pallas_call(
        _prep_body,
        grid=(b // bt,),
        in_specs=[
            pl.BlockSpec((bt, d), lambda i: (i, 0)),
            pl.BlockSpec((bt, d), lambda i: (i, 0)),
        ],
        out_specs=[
            pl.BlockSpec((d, bt), lambda i: (0, i)),
            pl.BlockSpec((d, bt), lambda i: (0, i)),
        ],
        out_shape=[
            jax.ShapeDtypeStruct((d, b), jnp.int32),
            jax.ShapeDtypeStruct((d, b), jnp.float32),
        ],
    )
    row_t, src_t = prep(idx32, src)

    total = d * b
    sc_scatter = _make_sc_scatter(m, d, b, 32)
    return sc_scatter(x, row_t.reshape(total), src_t.reshape(total))


# speedup vs baseline: 2.1715x; 2.1715x over previous
import functools

import jax
import jax.numpy as jnp
from jax import lax
from jax.experimental import pallas as pl
from jax.experimental.pallas import tpu as pltpu
from jax.experimental.pallas import tpu_sc as plsc


def _prep_body(idx_ref, src_ref, rowT_ref, srcT_ref):
    rowT_ref[...] = idx_ref[...].T
    srcT_ref[...] = src_ref[...].T


_GROUPS = 8
_GCOLS = 16
_CHUNK = 4096


def _make_sc_scatter(m, d, b, nw):
    mesh = plsc.VectorSubcoreMesh(core_axis_name="c", subcore_axis_name="s")
    bands = 20
    rows_band = m // bands
    n_tiles = _GROUPS * bands
    tiles_per_w = n_tiles // nw
    chunks_per_col = b // _CHUNK
    n_chunks = _GCOLS * chunks_per_col
    nv = _CHUNK // 16

    @functools.partial(
        pl.kernel,
        mesh=mesh,
        out_type=jax.ShapeDtypeStruct((m, d), jnp.float32),
        compiler_params=pltpu.CompilerParams(
            needs_layout_passes=False, use_tc_tiling_on_sc=False,
            disable_bounds_checks=True),
        scratch_types=[
            pltpu.VMEM((rows_band, _GCOLS), jnp.float32),
            pltpu.VMEM((_CHUNK,), jnp.int32),
            pltpu.VMEM((_CHUNK,), jnp.int32),
            pltpu.VMEM((_CHUNK,), jnp.float32),
            pltpu.VMEM((_CHUNK,), jnp.float32),
            pltpu.SemaphoreType.DMA,
            pltpu.SemaphoreType.DMA,
            pltpu.SemaphoreType.DMA,
            pltpu.SemaphoreType.DMA,
        ],
    )
    def _sc(x_ref, rowT_ref, srcT_ref, out_ref,
            tile_v, rb0, rb1, vb0, vb1, sr0, sr1, sv0, sv1):
        c = lax.axis_index("c")
        s = lax.axis_index("s")
        w = s * 2 + c

        def chunk_off(g, q):
            col = q // chunks_per_col
            hc = lax.rem(q, chunks_per_col)
            return (g * _GCOLS + col) * b + hc * _CHUNK

        for i in range(tiles_per_w):
            tid = w + nw * i
            g = lax.rem(tid, _GROUPS)
            band = tid // _GROUPS
            b0 = band * rows_band
            c0 = g * _GCOLS

            if True:
                pltpu.sync_copy(
                    x_ref.at[pl.ds(b0, rows_band), pl.ds(c0, _GCOLS)], tile_v)

            pltpu.async_copy(rowT_ref.at[pl.ds(chunk_off(g, 0), _CHUNK)],
                             rb0, sr0)
            pltpu.async_copy(srcT_ref.at[pl.ds(chunk_off(g, 0), _CHUNK)],
                             vb0, sv0)
            pltpu.async_copy(rowT_ref.at[pl.ds(chunk_off(g, 1), _CHUNK)],
                             rb1, sr1)
            pltpu.async_copy(srcT_ref.at[pl.ds(chunk_off(g, 1), _CHUNK)],
                             vb1, sv1)

            @pl.loop(0, n_chunks // 2)
            def _pair(p):
                for ph, (rb, vb, sr, sv) in enumerate(
                        ((rb0, vb0, sr0, sv0), (rb1, vb1, sr1, sv1))):
                    q = 2 * p + ph
                    col = q // chunks_per_col
                    pltpu.make_async_copy(
                        rowT_ref.at[pl.ds(0, _CHUNK)], rb, sr).wait()
                    pltpu.make_async_copy(
                        srcT_ref.at[pl.ds(0, _CHUNK)], vb, sv).wait()

                    @pl.loop(0, nv // 8)
                    def _vec(tt):
                        rs, vs = [], []
                        for j in range(8):
                            sl = pl.ds((tt * 8 + j) * 16, 16)
                            rs.append(rb[sl])
                            vs.append(vb[sl])
                        c16 = jnp.full((16,), col, jnp.int32)
                        for j in range(8):
                            r16 = rs[j]
                            rr = r16 - b0
                            inb = rr.astype(jnp.uint32) < jnp.uint32(
                                rows_band)
                            plsc.store_scatter(
                                tile_v, [rr, c16], vs[j], mask=inb)

                    nq = q + 2

                    @pl.when(nq < n_chunks)
                    def _():
                        off = chunk_off(g, nq)
                        pltpu.async_copy(
                            rowT_ref.at[pl.ds(off, _CHUNK)], rb, sr)
                        pltpu.async_copy(
                            srcT_ref.at[pl.ds(off, _CHUNK)], vb, sv)

            if True:
                pltpu.sync_copy(
                    tile_v, out_ref.at[pl.ds(b0, rows_band), pl.ds(c0, _GCOLS)])

    return _sc


def kernel(x, index, src):
    m, d = x.shape
    b = index.shape[0]
    assert d == 128 and b % 128 == 0

    idx32 = index.astype(jnp.int32)

    bt = 2048
    prep = pl.pallas_call(
        _prep_body,
        grid=(b // bt,),
        in_specs=[
            pl.BlockSpec((bt, d), lambda i: (i, 0)),
            pl.BlockSpec((bt, d), lambda i: (i, 0)),
        ],
        out_specs=[
            pl.BlockSpec((d, bt), lambda i: (0, i)),
            pl.BlockSpec((d, bt), lambda i: (0, i)),
        ],
        out_shape=[
            jax.ShapeDtypeStruct((d, b), jnp.int32),
            jax.ShapeDtypeStruct((d, b), jnp.float32),
        ],
    )
    row_t, src_t = prep(idx32, src)

    total = d * b
    sc_scatter = _make_sc_scatter(m, d, b, 32)
    return sc_scatter(x, row_t.reshape(total), src_t.reshape(total))


# speedup vs baseline: 2.5048x; 1.1535x over previous
import functools

import jax
import jax.numpy as jnp
from jax import lax
from jax.experimental import pallas as pl
from jax.experimental.pallas import tpu as pltpu
from jax.experimental.pallas import tpu_sc as plsc


def _prep_body(idx_ref, src_ref, rowT_ref, srcT_ref):
    rowT_ref[...] = idx_ref[...].T
    srcT_ref[...] = src_ref[...].T


_GROUPS = 8
_GCOLS = 16
_CHUNK = 8192


def _make_sc_scatter(m, d, b, nw):
    mesh = plsc.VectorSubcoreMesh(core_axis_name="c", subcore_axis_name="s")
    bands = 20
    rows_band = m // bands
    n_tiles = _GROUPS * bands
    tiles_per_w = n_tiles // nw
    chunks_per_col = b // _CHUNK
    n_chunks = _GCOLS * chunks_per_col
    nv = _CHUNK // 16

    @functools.partial(
        pl.kernel,
        mesh=mesh,
        out_type=jax.ShapeDtypeStruct((m, d), jnp.float32),
        compiler_params=pltpu.CompilerParams(
            needs_layout_passes=False, use_tc_tiling_on_sc=False,
            disable_bounds_checks=True),
        scratch_types=[
            pltpu.VMEM((rows_band, _GCOLS), jnp.float32),
            pltpu.VMEM((_CHUNK,), jnp.int32),
            pltpu.VMEM((_CHUNK,), jnp.int32),
            pltpu.VMEM((_CHUNK,), jnp.float32),
            pltpu.VMEM((_CHUNK,), jnp.float32),
            pltpu.SemaphoreType.DMA,
            pltpu.SemaphoreType.DMA,
            pltpu.SemaphoreType.DMA,
            pltpu.SemaphoreType.DMA,
        ],
    )
    def _sc(x_ref, rowT_ref, srcT_ref, out_ref,
            tile_v, rb0, rb1, vb0, vb1, sr0, sr1, sv0, sv1):
        c = lax.axis_index("c")
        s = lax.axis_index("s")
        w = s * 2 + c

        def chunk_off(g, q):
            col = q // chunks_per_col
            hc = lax.rem(q, chunks_per_col)
            return (g * _GCOLS + col) * b + hc * _CHUNK

        for i in range(tiles_per_w):
            tid = w + nw * i
            g = lax.rem(tid, _GROUPS)
            band = tid // _GROUPS
            b0 = band * rows_band
            c0 = g * _GCOLS

            if True:
                pltpu.sync_copy(
                    x_ref.at[pl.ds(b0, rows_band), pl.ds(c0, _GCOLS)], tile_v)

            pltpu.async_copy(rowT_ref.at[pl.ds(chunk_off(g, 0), _CHUNK)],
                             rb0, sr0)
            pltpu.async_copy(srcT_ref.at[pl.ds(chunk_off(g, 0), _CHUNK)],
                             vb0, sv0)
            pltpu.async_copy(rowT_ref.at[pl.ds(chunk_off(g, 1), _CHUNK)],
                             rb1, sr1)
            pltpu.async_copy(srcT_ref.at[pl.ds(chunk_off(g, 1), _CHUNK)],
                             vb1, sv1)

            @pl.loop(0, n_chunks // 2)
            def _pair(p):
                for ph, (rb, vb, sr, sv) in enumerate(
                        ((rb0, vb0, sr0, sv0), (rb1, vb1, sr1, sv1))):
                    q = 2 * p + ph
                    col = q // chunks_per_col
                    pltpu.make_async_copy(
                        rowT_ref.at[pl.ds(0, _CHUNK)], rb, sr).wait()
                    pltpu.make_async_copy(
                        srcT_ref.at[pl.ds(0, _CHUNK)], vb, sv).wait()

                    @pl.loop(0, nv // 8)
                    def _vec(tt):
                        rs, vs = [], []
                        for j in range(8):
                            sl = pl.ds((tt * 8 + j) * 16, 16)
                            rs.append(rb[sl])
                            vs.append(vb[sl])
                        c16 = jnp.full((16,), col, jnp.int32)
                        for j in range(8):
                            r16 = rs[j]
                            rr = r16 - b0
                            inb = rr.astype(jnp.uint32) < jnp.uint32(
                                rows_band)
                            plsc.store_scatter(
                                tile_v, [rr, c16], vs[j], mask=inb)

                    nq = q + 2

                    @pl.when(nq < n_chunks)
                    def _():
                        off = chunk_off(g, nq)
                        pltpu.async_copy(
                            rowT_ref.at[pl.ds(off, _CHUNK)], rb, sr)
                        pltpu.async_copy(
                            srcT_ref.at[pl.ds(off, _CHUNK)], vb, sv)

            if True:
                pltpu.sync_copy(
                    tile_v, out_ref.at[pl.ds(b0, rows_band), pl.ds(c0, _GCOLS)])

    return _sc


def kernel(x, index, src):
    m, d = x.shape
    b = index.shape[0]
    assert d == 128 and b % 128 == 0

    idx32 = index.astype(jnp.int32)

    bt = 2048
    prep = pl.pallas_call(
        _prep_body,
        grid=(b // bt,),
        in_specs=[
            pl.BlockSpec((bt, d), lambda i: (i, 0)),
            pl.BlockSpec((bt, d), lambda i: (i, 0)),
        ],
        out_specs=[
            pl.BlockSpec((d, bt), lambda i: (0, i)),
            pl.BlockSpec((d, bt), lambda i: (0, i)),
        ],
        out_shape=[
            jax.ShapeDtypeStruct((d, b), jnp.int32),
            jax.ShapeDtypeStruct((d, b), jnp.float32),
        ],
    )
    row_t, src_t = prep(idx32, src)

    total = d * b
    sc_scatter = _make_sc_scatter(m, d, b, 32)
    return sc_scatter(x, row_t.reshape(total), src_t.reshape(total))


# speedup vs baseline: 2.5806x; 1.0303x over previous
import functools

import jax
import jax.numpy as jnp
from jax import lax
from jax.experimental import pallas as pl
from jax.experimental.pallas import tpu as pltpu
from jax.experimental.pallas import tpu_sc as plsc


def _prep_body(idx_ref, src_ref, rowT_ref, srcT_ref):
    rowT_ref[...] = idx_ref[...].T
    srcT_ref[...] = src_ref[...].T


_GROUPS = 8
_GCOLS = 16
_CHUNK = 8192


def _make_sc_scatter(m, d, b, nw):
    mesh = plsc.VectorSubcoreMesh(core_axis_name="c", subcore_axis_name="s")
    bands = 20
    rows_band = m // bands
    n_tiles = _GROUPS * bands
    tiles_per_w = n_tiles // nw
    chunks_per_col = b // _CHUNK
    n_chunks = _GCOLS * chunks_per_col
    nv = _CHUNK // 16

    @functools.partial(
        pl.kernel,
        mesh=mesh,
        out_type=jax.ShapeDtypeStruct((m, d), jnp.float32),
        compiler_params=pltpu.CompilerParams(
            needs_layout_passes=False, use_tc_tiling_on_sc=False,
            disable_bounds_checks=True),
        scratch_types=[
            pltpu.VMEM((rows_band, _GCOLS), jnp.float32),
            pltpu.VMEM((_CHUNK,), jnp.int32),
            pltpu.VMEM((_CHUNK,), jnp.int32),
            pltpu.VMEM((_CHUNK,), jnp.float32),
            pltpu.VMEM((_CHUNK,), jnp.float32),
            pltpu.SemaphoreType.DMA,
            pltpu.SemaphoreType.DMA,
            pltpu.SemaphoreType.DMA,
            pltpu.SemaphoreType.DMA,
        ],
    )
    def _sc(x_ref, rowT_ref, srcT_ref, out_ref,
            tile_v, rb0, rb1, vb0, vb1, sr0, sr1, sv0, sv1):
        c = lax.axis_index("c")
        s = lax.axis_index("s")
        w = s * 2 + c

        def chunk_off(g, q):
            col = q // chunks_per_col
            hc = lax.rem(q, chunks_per_col)
            return (g * _GCOLS + col) * b + hc * _CHUNK

        def prime(g):
            pltpu.async_copy(rowT_ref.at[pl.ds(chunk_off(g, 0), _CHUNK)],
                             rb0, sr0)
            pltpu.async_copy(srcT_ref.at[pl.ds(chunk_off(g, 0), _CHUNK)],
                             vb0, sv0)
            pltpu.async_copy(rowT_ref.at[pl.ds(chunk_off(g, 1), _CHUNK)],
                             rb1, sr1)
            pltpu.async_copy(srcT_ref.at[pl.ds(chunk_off(g, 1), _CHUNK)],
                             vb1, sv1)

        def tgeom(i):
            tid = w + nw * i
            g = lax.rem(tid, _GROUPS)
            band = tid // _GROUPS
            return g, band * rows_band, g * _GCOLS

        prime(tgeom(0)[0])
        for i in range(tiles_per_w):
            g, b0, c0 = tgeom(i)

            pltpu.sync_copy(
                x_ref.at[pl.ds(b0, rows_band), pl.ds(c0, _GCOLS)], tile_v)

            @pl.loop(0, n_chunks // 2)
            def _pair(p):
                for ph, (rb, vb, sr, sv) in enumerate(
                        ((rb0, vb0, sr0, sv0), (rb1, vb1, sr1, sv1))):
                    q = 2 * p + ph
                    col = q // chunks_per_col
                    pltpu.make_async_copy(
                        rowT_ref.at[pl.ds(0, _CHUNK)], rb, sr).wait()
                    pltpu.make_async_copy(
                        srcT_ref.at[pl.ds(0, _CHUNK)], vb, sv).wait()

                    @pl.loop(0, nv // 8)
                    def _vec(tt):
                        rs, vs = [], []
                        for j in range(8):
                            sl = pl.ds((tt * 8 + j) * 16, 16)
                            rs.append(rb[sl])
                            vs.append(vb[sl])
                        c16 = jnp.full((16,), col, jnp.int32)
                        for j in range(8):
                            r16 = rs[j]
                            rr = r16 - b0
                            inb = rr.astype(jnp.uint32) < jnp.uint32(
                                rows_band)
                            plsc.store_scatter(
                                tile_v, [rr, c16], vs[j], mask=inb)

                    nq = q + 2

                    @pl.when(nq < n_chunks)
                    def _():
                        off = chunk_off(g, nq)
                        pltpu.async_copy(
                            rowT_ref.at[pl.ds(off, _CHUNK)], rb, sr)
                        pltpu.async_copy(
                            srcT_ref.at[pl.ds(off, _CHUNK)], vb, sv)

            if i + 1 < tiles_per_w:
                prime(tgeom(i + 1)[0])
            pltpu.sync_copy(
                tile_v, out_ref.at[pl.ds(b0, rows_band), pl.ds(c0, _GCOLS)])

    return _sc


def kernel(x, index, src):
    m, d = x.shape
    b = index.shape[0]
    assert d == 128 and b % 128 == 0

    idx32 = index.astype(jnp.int32)

    bt = 2048
    prep = pl.pallas_call(
        _prep_body,
        grid=(b // bt,),
        in_specs=[
            pl.BlockSpec((bt, d), lambda i: (i, 0)),
            pl.BlockSpec((bt, d), lambda i: (i, 0)),
        ],
        out_specs=[
            pl.BlockSpec((d, bt), lambda i: (0, i)),
            pl.BlockSpec((d, bt), lambda i: (0, i)),
        ],
        out_shape=[
            jax.ShapeDtypeStruct((d, b), jnp.int32),
            jax.ShapeDtypeStruct((d, b), jnp.float32),
        ],
    )
    row_t, src_t = prep(idx32, src)

    total = d * b
    sc_scatter = _make_sc_scatter(m, d, b, 32)
    return sc_scatter(x, row_t.reshape(total), src_t.reshape(total))


# speedup vs baseline: 2.8707x; 1.1124x over previous
import functools

import jax
import jax.numpy as jnp
from jax import lax
from jax.experimental import pallas as pl
from jax.experimental.pallas import tpu as pltpu
from jax.experimental.pallas import tpu_sc as plsc


def _prep_body(idx_ref, src_ref, rowT_ref, srcT_ref):
    rowT_ref[...] = idx_ref[...].T
    srcT_ref[...] = src_ref[...].T


_GROUPS = 16
_GCOLS = 8
_CHUNK = 8192


def _make_sc_scatter(m, d, b, nw):
    mesh = plsc.VectorSubcoreMesh(core_axis_name="c", subcore_axis_name="s")
    bands = 10
    rows_band = m // bands
    n_tiles = _GROUPS * bands
    tiles_per_w = n_tiles // nw
    chunks_per_col = b // _CHUNK
    n_chunks = _GCOLS * chunks_per_col
    nv = _CHUNK // 16

    @functools.partial(
        pl.kernel,
        mesh=mesh,
        out_type=jax.ShapeDtypeStruct((m, d), jnp.float32),
        compiler_params=pltpu.CompilerParams(
            needs_layout_passes=False, use_tc_tiling_on_sc=False,
            disable_bounds_checks=True),
        scratch_types=[
            pltpu.VMEM((rows_band, _GCOLS), jnp.float32),
            pltpu.VMEM((_CHUNK,), jnp.int32),
            pltpu.VMEM((_CHUNK,), jnp.int32),
            pltpu.VMEM((_CHUNK,), jnp.float32),
            pltpu.VMEM((_CHUNK,), jnp.float32),
            pltpu.SemaphoreType.DMA,
            pltpu.SemaphoreType.DMA,
            pltpu.SemaphoreType.DMA,
            pltpu.SemaphoreType.DMA,
        ],
    )
    def _sc(x_ref, rowT_ref, srcT_ref, out_ref,
            tile_v, rb0, rb1, vb0, vb1, sr0, sr1, sv0, sv1):
        c = lax.axis_index("c")
        s = lax.axis_index("s")
        w = s * 2 + c

        def chunk_off(g, q):
            col = q // chunks_per_col
            hc = lax.rem(q, chunks_per_col)
            return (g * _GCOLS + col) * b + hc * _CHUNK

        def prime(g):
            pltpu.async_copy(rowT_ref.at[pl.ds(chunk_off(g, 0), _CHUNK)],
                             rb0, sr0)
            pltpu.async_copy(srcT_ref.at[pl.ds(chunk_off(g, 0), _CHUNK)],
                             vb0, sv0)
            pltpu.async_copy(rowT_ref.at[pl.ds(chunk_off(g, 1), _CHUNK)],
                             rb1, sr1)
            pltpu.async_copy(srcT_ref.at[pl.ds(chunk_off(g, 1), _CHUNK)],
                             vb1, sv1)

        def tgeom(i):
            tid = w + nw * i
            g = lax.rem(tid, _GROUPS)
            band = tid // _GROUPS
            return g, band * rows_band, g * _GCOLS

        prime(tgeom(0)[0])
        for i in range(tiles_per_w):
            g, b0, c0 = tgeom(i)

            pltpu.sync_copy(
                x_ref.at[pl.ds(b0, rows_band), pl.ds(c0, _GCOLS)], tile_v)

            @pl.loop(0, n_chunks // 2)
            def _pair(p):
                for ph, (rb, vb, sr, sv) in enumerate(
                        ((rb0, vb0, sr0, sv0), (rb1, vb1, sr1, sv1))):
                    q = 2 * p + ph
                    col = q // chunks_per_col
                    pltpu.make_async_copy(
                        rowT_ref.at[pl.ds(0, _CHUNK)], rb, sr).wait()
                    pltpu.make_async_copy(
                        srcT_ref.at[pl.ds(0, _CHUNK)], vb, sv).wait()

                    @pl.loop(0, nv // 8)
                    def _vec(tt):
                        rs, vs = [], []
                        for j in range(8):
                            sl = pl.ds((tt * 8 + j) * 16, 16)
                            rs.append(rb[sl])
                            vs.append(vb[sl])
                        c16 = jnp.full((16,), col, jnp.int32)
                        for j in range(8):
                            r16 = rs[j]
                            rr = r16 - b0
                            inb = rr.astype(jnp.uint32) < jnp.uint32(
                                rows_band)
                            plsc.store_scatter(
                                tile_v, [rr, c16], vs[j], mask=inb)

                    nq = q + 2

                    @pl.when(nq < n_chunks)
                    def _():
                        off = chunk_off(g, nq)
                        pltpu.async_copy(
                            rowT_ref.at[pl.ds(off, _CHUNK)], rb, sr)
                        pltpu.async_copy(
                            srcT_ref.at[pl.ds(off, _CHUNK)], vb, sv)

            if i + 1 < tiles_per_w:
                prime(tgeom(i + 1)[0])
            pltpu.sync_copy(
                tile_v, out_ref.at[pl.ds(b0, rows_band), pl.ds(c0, _GCOLS)])

    return _sc


def kernel(x, index, src):
    m, d = x.shape
    b = index.shape[0]
    assert d == 128 and b % 128 == 0

    idx32 = index.astype(jnp.int32)

    bt = 2048
    prep = pl.pallas_call(
        _prep_body,
        grid=(b // bt,),
        in_specs=[
            pl.BlockSpec((bt, d), lambda i: (i, 0)),
            pl.BlockSpec((bt, d), lambda i: (i, 0)),
        ],
        out_specs=[
            pl.BlockSpec((d, bt), lambda i: (0, i)),
            pl.BlockSpec((d, bt), lambda i: (0, i)),
        ],
        out_shape=[
            jax.ShapeDtypeStruct((d, b), jnp.int32),
            jax.ShapeDtypeStruct((d, b), jnp.float32),
        ],
    )
    row_t, src_t = prep(idx32, src)

    total = d * b
    sc_scatter = _make_sc_scatter(m, d, b, 32)
    return sc_scatter(x, row_t.reshape(total), src_t.reshape(total))
